# baseline (speedup 1.0000x reference)
import functools

import jax
import jax.numpy as jnp
from jax import lax
from jax.experimental import pallas as pl
from jax.experimental.pallas import tpu as pltpu
from jax.experimental.pallas import tpu_sc as plsc

B = 16384
D = 64
NH = 128
NW = 32
RPW = B // NW
NCHUNK = 4
CH = RPW // NCHUNK
L = 16
GPC = CH // L


def _sc_kernel_body(tidx_hbm, midx_hbm, tt_hbm, mt_hbm, w_hbm, out_hbm,
                    idx_t, idx_m, et, em, wv, ov, *sems):
    wid = lax.axis_index("s") * 2 + lax.axis_index("c")
    pltpu.sync_copy(tidx_hbm.at[wid], idx_t)
    pltpu.sync_copy(midx_hbm.at[wid], idx_m)
    pltpu.sync_copy(w_hbm, wv)

    copies = []
    for c in range(NCHUNK):
        copies.append(pltpu.async_copy(
            tt_hbm.at[idx_t.at[c]], et.at[pl.ds(c * CH, CH)], sems[2 * c]))
        copies.append(pltpu.async_copy(
            mt_hbm.at[idx_m.at[c]], em.at[pl.ds(c * CH, CH)], sems[2 * c + 1]))

    lane = lax.iota(jnp.int32, 16)
    lane3 = lane * 3

    for c in range(NCHUNK):
        copies[2 * c].wait()
        copies[2 * c + 1].wait()

        def group_body(g, carry):
            row = lane + g * L
            acc0 = jnp.zeros((L,), jnp.float32)
            acc1 = jnp.zeros((L,), jnp.float32)
            acc2 = jnp.zeros((L,), jnp.float32)
            for d in range(D):
                col = jnp.full((L,), d, jnp.int32)
                etv = plsc.load_gather(et, [row, col])
                emv = plsc.load_gather(em, [row, col])
                pv = etv * emv
                wrow = wv[d]
                acc0 = acc0 + etv * wrow[0] + emv * wrow[3] + pv * wrow[6]
                acc1 = acc1 + etv * wrow[1] + emv * wrow[4] + pv * wrow[7]
                acc2 = acc2 + etv * wrow[2] + emv * wrow[5] + pv * wrow[8]
            ob = g * (3 * L)
            plsc.store_scatter(ov, [lane3 + ob], acc0)
            plsc.store_scatter(ov, [lane3 + (ob + 1)], acc1)
            plsc.store_scatter(ov, [lane3 + (ob + 2)], acc2)
            return carry

        lax.fori_loop(c * GPC, (c + 1) * GPC, group_body, 0)

    pltpu.sync_copy(ov, out_hbm.at[pl.ds(wid * (3 * RPW), 3 * RPW)])


@jax.jit
def _sc_part(tidx, midx, teacher_table, materia_table, w_emb):
    mesh = plsc.VectorSubcoreMesh(core_axis_name="c", subcore_axis_name="s")
    scratch = [
        pltpu.VMEM((NCHUNK, CH), jnp.int32),
        pltpu.VMEM((NCHUNK, CH), jnp.int32),
        pltpu.VMEM((RPW, D), jnp.float32),
        pltpu.VMEM((RPW, D), jnp.float32),
        pltpu.VMEM((D, L), jnp.float32),
        pltpu.VMEM((3 * RPW,), jnp.float32),
    ] + [pltpu.SemaphoreType.DMA] * (2 * NCHUNK)
    return pl.kernel(
        _sc_kernel_body,
        mesh=mesh,
        out_type=jax.ShapeDtypeStruct((B * 3,), jnp.float32),
        scratch_types=scratch,
        compiler_params=pltpu.CompilerParams(
            needs_layout_passes=False, use_tc_tiling_on_sc=False),
    )(tidx, midx, teacher_table, materia_table, w_emb)


def _tc_body(h_ref, w_ref, b_ref, o_ref):
    o_ref[...] = jnp.dot(h_ref[...], w_ref[...],
                         preferred_element_type=jnp.float32) + b_ref[...]


@jax.jit
def _tc_part(h, w_h, b2):
    blk = 2048
    return pl.pallas_call(
        _tc_body,
        grid=(B // blk,),
        in_specs=[
            pl.BlockSpec((blk, NH), lambda i: (i, 0)),
            pl.BlockSpec((NH, 3), lambda i: (0, 0)),
            pl.BlockSpec((1, 3), lambda i: (0, 0)),
        ],
        out_specs=pl.BlockSpec((blk, 3), lambda i: (i, 0)),
        out_shape=jax.ShapeDtypeStruct((B, 3), jnp.float32),
    )(h, w_h, b2)


def kernel(h, teacher_idx, materia_idx, teacher_table, materia_table, W, b):
    tidx = teacher_idx.astype(jnp.int32).reshape(NW, NCHUNK, CH)
    midx = materia_idx.astype(jnp.int32).reshape(NW, NCHUNK, CH)
    w_pack = jnp.concatenate(
        [W[NH:NH + D], W[NH + D:NH + 2 * D], W[NH + 2 * D:],
         jnp.zeros((D, 7), jnp.float32)], axis=1)
    sc = _sc_part(tidx, midx, teacher_table, materia_table, w_pack)
    tc = _tc_part(h, W[:NH], b.reshape(1, 3))
    return tc + sc.reshape(B, 3)
